# baseline (speedup 1.0000x reference)
import functools

import jax
import jax.numpy as jnp
from jax import lax
from jax.experimental import pallas as pl
from jax.experimental.pallas import tpu as pltpu
from jax.experimental.pallas import tpu_sc as plsc

NC = 2
NS = 16
NW = NC * NS
CHUNK = 128


def _round_up(a, b):
    return (a + b - 1) // b * b


NB = 2


def _make_scatter_kernel(n_pad, e_pad, widths, n_tables, split=None):
    cps = e_pad // (NS * CHUNK)
    if split is None:
        split = (cps // 2, cps - cps // 2)
    cpt0, cpt1 = split
    assert cpt0 + cpt1 == cps and cpt0 % NB == 0 and cpt1 % NB == 0
    wave = cps // 2
    n_waves = -(-max(cpt0, cpt1) // wave)
    zpt = n_pad // (NS * CHUNK)
    n_str = len(widths)

    mesh = plsc.VectorSubcoreMesh(
        core_axis_name="c", subcore_axis_name="s",
        num_cores=NC, num_subcores=NS)

    out_type = tuple(
        jax.ShapeDtypeStruct((NC, n_pad, w), jnp.float32) for w in widths)
    scratch = (
        [pltpu.VMEM((wave, CHUNK), jnp.int32),
         pltpu.VMEM((wave, CHUNK), jnp.int32)]
        + [pltpu.VMEM((NB, CHUNK, w), jnp.float32) for w in widths]
        + [pltpu.VMEM((CHUNK, w), jnp.float32) for w in widths]
        + [pltpu.VMEM_SHARED((n_pad, w), jnp.float32) for w in widths]
        + [pltpu.SemaphoreType.DMA, pltpu.SemaphoreType.DMA]
    )

    @functools.partial(
        pl.kernel, out_type=out_type, mesh=mesh, scratch_types=scratch,
        compiler_params=pltpu.CompilerParams(use_tc_tiling_on_sc=False))
    def kern(src_hbm, dst_hbm, *rest):
        tabs = rest[:n_str]
        zeros_hbm = rest[n_str:2 * n_str]
        outs = rest[2 * n_str:3 * n_str]
        idx_s = rest[3 * n_str]
        idx_d = rest[3 * n_str + 1]
        rows = rest[3 * n_str + 2:4 * n_str + 2]
        zbufs = rest[4 * n_str + 2:5 * n_str + 2]
        shs = rest[5 * n_str + 2:6 * n_str + 2]
        sem_g = rest[6 * n_str + 2]
        sem_z = rest[6 * n_str + 3]

        cid = lax.axis_index("c")
        sid = lax.axis_index("s")
        my_cpt = jnp.where(cid == 0, cpt0, cpt1)
        chunk0 = sid * cps + cid * cpt0

        def stage_idx_wave(cb, cw):
            def stage(i, c):
                base = (cb + i) * CHUNK
                pltpu.async_copy(src_hbm.at[pl.ds(base, CHUNK)], idx_s.at[i],
                                 sem_z)
                pltpu.async_copy(dst_hbm.at[pl.ds(base, CHUNK)], idx_d.at[i],
                                 sem_z)
                return c
            lax.fori_loop(0, cw, stage, 0)

        def drain_idx_wave(cb, cw):
            def drain(i, c):
                base = (cb + i) * CHUNK
                pltpu.make_async_copy(src_hbm.at[pl.ds(base, CHUNK)],
                                      idx_s.at[i], sem_z).wait()
                pltpu.make_async_copy(dst_hbm.at[pl.ds(base, CHUNK)],
                                      idx_d.at[i], sem_z).wait()
                return c
            lax.fori_loop(0, cw, drain, 0)

        def start_gathers(cb, i, b):
            for t in range(n_str):
                if t < n_tables:
                    pltpu.async_copy(tabs[t].at[idx_s.at[i]], rows[t].at[b],
                                     sem_g)
                else:
                    base = (cb + i) * CHUNK
                    pltpu.async_copy(tabs[t].at[pl.ds(base, CHUNK)],
                                     rows[t].at[b], sem_g)

        def wait_gathers(cb, i, b):
            for t in range(n_str):
                if t < n_tables:
                    pltpu.make_async_copy(tabs[t].at[idx_s.at[i]],
                                          rows[t].at[b], sem_g).wait()
                else:
                    base = (cb + i) * CHUNK
                    pltpu.make_async_copy(tabs[t].at[pl.ds(base, CHUNK)],
                                          rows[t].at[b], sem_g).wait()

        def gather_scatter_wave(cb, cw):
            for b in range(NB):
                @pl.when(b < cw)
                def _(b=b):
                    start_gathers(cb, b, b)

            def body(o, carry):
                for b in range(NB):
                    i = o * NB + b
                    wait_gathers(cb, i, b)
                    for t in range(n_str):
                        pltpu.sync_copy(rows[t].at[b], shs[t].at[idx_d.at[i]],
                                        add=True)

                    @pl.when(i + NB < cw)
                    def _():
                        start_gathers(cb, i + NB, b)
                return carry

            lax.fori_loop(0, cw // NB, body, 0)

        def wave_args(w):
            cb = chunk0 + w * wave
            cw = jnp.clip(my_cpt - w * wave, 0, wave)
            return cb, cw

        stage_idx_wave(*wave_args(0))

        @pl.when(my_cpt > 0)
        def _():
            for t in range(n_str):
                pltpu.sync_copy(zeros_hbm[t], zbufs[t])
            for z in range(zpt):
                row0 = (sid * zpt + z) * CHUNK
                for t in range(n_str):
                    pltpu.sync_copy(zbufs[t], shs[t].at[pl.ds(row0, CHUNK)])

        drain_idx_wave(*wave_args(0))
        plsc.subcore_barrier()
        gather_scatter_wave(*wave_args(0))
        for w in range(1, n_waves):
            cb, cw = wave_args(w)
            stage_idx_wave(cb, cw)
            drain_idx_wave(cb, cw)
            gather_scatter_wave(cb, cw)
        plsc.subcore_barrier()

        @pl.when(my_cpt > 0)
        def _():
            for z in range(zpt):
                row0 = (sid * zpt + z) * CHUNK
                for t in range(n_str):
                    pltpu.sync_copy(shs[t].at[pl.ds(row0, CHUNK)],
                                    rows[t].at[0])
                    pltpu.sync_copy(rows[t].at[0],
                                    outs[t].at[cid, pl.ds(row0, CHUNK)])

    return kern


def _tc1_body(x, xs_c, ea_c,
              w_nb, b_nb, w_nu0, b_nu0, w_nu1, b_nu1, w_nu2, b_nu2,
              w_ed, b_ed, w_eu, b_eu, w_2nb, b_2nb,
              x1_o, a_o, rinv_o, inv_o):
    f32 = jnp.float32
    dot = functools.partial(jnp.dot, preferred_element_type=f32)
    ea = ea_c[0] + ea_c[1]
    ea7 = ea[:, 0:7]
    cnt = ea[:, 7:8]
    inv = 1.0 / jnp.maximum(cnt, 1.0)
    xs = (xs_c[0] + xs_c[1])[:, 0:3]
    agg1 = (dot(xs, w_nb[0:3, :]) + dot(ea7, w_nb[3:10, :])
            + cnt * b_nb[...]) * inv
    z = jnp.maximum(dot(x[...], w_nu0[0:3, :]) + dot(agg1, w_nu0[3:45, :])
                    + b_nu0[...], 0.0)
    z = jnp.maximum(dot(z, w_nu1[...]) + b_nu1[...], 0.0)
    x1 = dot(z, w_nu2[...]) + b_nu2[...]
    wa = w_eu[0:128, :]
    wb = w_eu[128:256, :]
    wc = w_eu[256:384, :]
    w2a = w_2nb[0:128, :]
    w2b = w_2nb[128:256, :]
    wa_eff = w2a + dot(wa, w2b)
    wb_eff = dot(wb, w2b)
    k_eff = dot(dot(w_ed[...], wc), w2b)
    c2 = dot(dot(b_ed[...], wc) + b_eu[...], w2b) + b_2nb[...]
    a = dot(x1, wa_eff)
    b = dot(x1, wb_eff)
    rinv = (cnt * (b + c2) + dot(ea7, k_eff)) * inv
    x1_o[...] = x1
    a_o[...] = a
    rinv_o[...] = rinv
    inv_o[...] = inv


def _tc2_body(x1, as_c, rinv, inv, w_nu0, b_nu0, w_nu1, b_nu1,
              w_nu2, b_nu2, out_o):
    f32 = jnp.float32
    dot = functools.partial(jnp.dot, preferred_element_type=f32)
    agg2 = as_c[0] * inv[...] + rinv[...]
    z = jnp.maximum(dot(x1[...], w_nu0[0:128, :]) + dot(agg2, w_nu0[128:240, :])
                    + b_nu0[...], 0.0)
    z = jnp.maximum(dot(z, w_nu1[...]) + b_nu1[...], 0.0)
    out_o[...] = dot(z, w_nu2[...]) + b_nu2[...]


def _row_spec(r, w):
    return pl.BlockSpec((r, w), lambda i: (i, 0))


def _full_spec(shape):
    nd = len(shape)
    return pl.BlockSpec(shape, lambda i: (0,) * nd)


def kernel(x, edge_index, edge_attr, params):
    n = x.shape[0]
    e = edge_index.shape[1]
    n_pad = _round_up(n + 1, NS * CHUNK)
    e_pad = _round_up(e, NW * CHUNK)

    src = edge_index[0]
    dst = edge_index[1]
    pad_e = e_pad - e
    src_p = jnp.concatenate([src, jnp.zeros((pad_e,), jnp.int32)])
    dst_p = jnp.concatenate(
        [dst, n + (jnp.arange(pad_e, dtype=jnp.int32) % (n_pad - n))])
    x8 = jnp.pad(x, ((0, 0), (0, 5)))
    ea8 = jnp.pad(
        jnp.concatenate([edge_attr, jnp.ones((e, 1), jnp.float32)], axis=1),
        ((0, pad_e), (0, 0)))

    sc_a = _make_scatter_kernel(n_pad, e_pad, widths=(8, 8), n_tables=1,
                                split=(46, 34))
    z8 = jnp.zeros((CHUNK, 8), jnp.float32)
    xs_c, ea_c = sc_a(src_p, dst_p, x8, ea8, z8, z8)

    (w_nb, b_nb), = params['l1_nb']
    (wn0, bn0), (wn1, bn1), (wn2, bn2) = params['l1_nu']
    (w_ed, b_ed), = params['l1_ed']
    (w_eu, b_eu), = params['l1_eu']
    (w_2nb, b_2nb), = params['l2_nb']
    r = 1000
    grid = n // r
    row2 = lambda w: _row_spec(r, w)
    stk = lambda w: pl.BlockSpec((2, r, w), lambda i: (0, i, 0))
    tc1 = pl.pallas_call(
        _tc1_body,
        grid=(grid,),
        in_specs=[row2(3), stk(8), stk(8)]
                 + [_full_spec(s.shape) for s in (
                     w_nb, b_nb.reshape(1, -1), wn0, bn0.reshape(1, -1),
                     wn1, bn1.reshape(1, -1), wn2, bn2.reshape(1, -1),
                     w_ed, b_ed.reshape(1, -1), w_eu, b_eu.reshape(1, -1),
                     w_2nb, b_2nb.reshape(1, -1))],
        out_specs=[row2(128), row2(112), row2(112), row2(1)],
        out_shape=[jax.ShapeDtypeStruct((n, 128), jnp.float32),
                   jax.ShapeDtypeStruct((n, 112), jnp.float32),
                   jax.ShapeDtypeStruct((n, 112), jnp.float32),
                   jax.ShapeDtypeStruct((n, 1), jnp.float32)],
    )
    x1, a_mat, rinv, inv = tc1(
        x, xs_c, ea_c,
        w_nb, b_nb.reshape(1, -1), wn0, bn0.reshape(1, -1),
        wn1, bn1.reshape(1, -1), wn2, bn2.reshape(1, -1),
        w_ed, b_ed.reshape(1, -1), w_eu, b_eu.reshape(1, -1),
        w_2nb, b_2nb.reshape(1, -1))

    sc_b = _make_scatter_kernel(n_pad, e_pad, widths=(112,), n_tables=1,
                                split=(80, 0))
    z112 = jnp.zeros((CHUNK, 112), jnp.float32)
    as_c, = sc_b(src_p, dst_p, a_mat, z112)

    (w2n0, b2n0), (w2n1, b2n1), (w2n2, b2n2) = params['l2_nu']
    tc2 = pl.pallas_call(
        _tc2_body,
        grid=(grid,),
        in_specs=[row2(128), pl.BlockSpec((1, r, 112), lambda i: (0, i, 0)),
                  row2(112), row2(1)]
                 + [_full_spec(s.shape) for s in (
                     w2n0, b2n0.reshape(1, -1), w2n1, b2n1.reshape(1, -1),
                     w2n2, b2n2.reshape(1, -1))],
        out_specs=[row2(264)],
        out_shape=[jax.ShapeDtypeStruct((n, 264), jnp.float32)],
    )
    x2, = tc2(x1, as_c, rinv, inv,
              w2n0, b2n0.reshape(1, -1), w2n1, b2n1.reshape(1, -1),
              w2n2, b2n2.reshape(1, -1))
    return x2


# speedup vs baseline: 1.1310x; 1.1310x over previous
import functools

import jax
import jax.numpy as jnp
from jax import lax
from jax.experimental import pallas as pl
from jax.experimental.pallas import tpu as pltpu
from jax.experimental.pallas import tpu_sc as plsc

NC = 2
NS = 16
NW = NC * NS
CHUNK = 128


def _round_up(a, b):
    return (a + b - 1) // b * b


NB = 2


def _make_scatter_kernel(n_pad, e_pad, widths, n_tables, split=None):
    cps = e_pad // (NS * CHUNK)
    if split is None:
        split = (cps // 2, cps - cps // 2)
    cpt0, cpt1 = split
    assert cpt0 + cpt1 == cps and cpt0 % NB == 0 and cpt1 % NB == 0
    wave = cps // 2
    n_waves = -(-max(cpt0, cpt1) // wave)
    zpt = n_pad // (NS * CHUNK)
    n_str = len(widths)

    mesh = plsc.VectorSubcoreMesh(
        core_axis_name="c", subcore_axis_name="s",
        num_cores=NC, num_subcores=NS)

    out_type = tuple(
        jax.ShapeDtypeStruct((NC, n_pad, w), jnp.float32) for w in widths)
    scratch = (
        [pltpu.VMEM((wave, CHUNK), jnp.int32),
         pltpu.VMEM((wave, CHUNK), jnp.int32)]
        + [pltpu.VMEM((NB, CHUNK, w), jnp.float32) for w in widths]
        + [pltpu.VMEM((CHUNK, w), jnp.float32) for w in widths]
        + [pltpu.VMEM_SHARED((n_pad, w), jnp.float32) for w in widths]
        + [pltpu.SemaphoreType.DMA, pltpu.SemaphoreType.DMA]
    )

    @functools.partial(
        pl.kernel, out_type=out_type, mesh=mesh, scratch_types=scratch,
        compiler_params=pltpu.CompilerParams(use_tc_tiling_on_sc=False))
    def kern(src_hbm, dst_hbm, *rest):
        tabs = rest[:n_str]
        zeros_hbm = rest[n_str:2 * n_str]
        outs = rest[2 * n_str:3 * n_str]
        idx_s = rest[3 * n_str]
        idx_d = rest[3 * n_str + 1]
        rows = rest[3 * n_str + 2:4 * n_str + 2]
        zbufs = rest[4 * n_str + 2:5 * n_str + 2]
        shs = rest[5 * n_str + 2:6 * n_str + 2]
        sem_g = rest[6 * n_str + 2]
        sem_z = rest[6 * n_str + 3]

        cid = lax.axis_index("c")
        sid = lax.axis_index("s")
        my_cpt = jnp.where(cid == 0, cpt0, cpt1)
        chunk0 = sid * cps + cid * cpt0

        def stage_idx_wave(cb, cw):
            def stage(i, c):
                base = (cb + i) * CHUNK
                pltpu.async_copy(src_hbm.at[pl.ds(base, CHUNK)], idx_s.at[i],
                                 sem_z)
                pltpu.async_copy(dst_hbm.at[pl.ds(base, CHUNK)], idx_d.at[i],
                                 sem_z)
                return c
            lax.fori_loop(0, cw, stage, 0)

        def drain_idx_wave(cb, cw):
            def drain(i, c):
                base = (cb + i) * CHUNK
                pltpu.make_async_copy(src_hbm.at[pl.ds(base, CHUNK)],
                                      idx_s.at[i], sem_z).wait()
                pltpu.make_async_copy(dst_hbm.at[pl.ds(base, CHUNK)],
                                      idx_d.at[i], sem_z).wait()
                return c
            lax.fori_loop(0, cw, drain, 0)

        def start_gathers(cb, i, b):
            for t in range(n_str):
                if t < n_tables:
                    pltpu.async_copy(tabs[t].at[idx_s.at[i]], rows[t].at[b],
                                     sem_g)
                else:
                    base = (cb + i) * CHUNK
                    pltpu.async_copy(tabs[t].at[pl.ds(base, CHUNK)],
                                     rows[t].at[b], sem_g)

        def wait_gathers(cb, i, b):
            for t in range(n_str):
                if t < n_tables:
                    pltpu.make_async_copy(tabs[t].at[idx_s.at[i]],
                                          rows[t].at[b], sem_g).wait()
                else:
                    base = (cb + i) * CHUNK
                    pltpu.make_async_copy(tabs[t].at[pl.ds(base, CHUNK)],
                                          rows[t].at[b], sem_g).wait()

        def gather_scatter_wave(cb, cw):
            for b in range(NB):
                @pl.when(b < cw)
                def _(b=b):
                    start_gathers(cb, b, b)

            def body(o, carry):
                for b in range(NB):
                    i = o * NB + b
                    wait_gathers(cb, i, b)
                    for t in range(n_str):
                        pltpu.sync_copy(rows[t].at[b], shs[t].at[idx_d.at[i]],
                                        add=True)

                    @pl.when(i + NB < cw)
                    def _():
                        start_gathers(cb, i + NB, b)
                return carry

            lax.fori_loop(0, cw // NB, body, 0)

        def wave_args(w):
            cb = chunk0 + w * wave
            cw = jnp.clip(my_cpt - w * wave, 0, wave)
            return cb, cw

        stage_idx_wave(*wave_args(0))

        @pl.when(my_cpt > 0)
        def _():
            for t in range(n_str):
                pltpu.sync_copy(zeros_hbm[t], zbufs[t])
            for z in range(zpt):
                row0 = (sid * zpt + z) * CHUNK
                for t in range(n_str):
                    pltpu.sync_copy(zbufs[t], shs[t].at[pl.ds(row0, CHUNK)])

        drain_idx_wave(*wave_args(0))
        plsc.subcore_barrier()
        gather_scatter_wave(*wave_args(0))
        for w in range(1, n_waves):
            cb, cw = wave_args(w)
            stage_idx_wave(cb, cw)
            drain_idx_wave(cb, cw)
            gather_scatter_wave(cb, cw)
        plsc.subcore_barrier()

        @pl.when(my_cpt > 0)
        def _():
            for z in range(zpt):
                row0 = (sid * zpt + z) * CHUNK
                for t in range(n_str):
                    pltpu.sync_copy(shs[t].at[pl.ds(row0, CHUNK)],
                                    rows[t].at[0])
                    pltpu.sync_copy(rows[t].at[0],
                                    outs[t].at[cid, pl.ds(row0, CHUNK)])

    return kern


def _tc1_body(x, xs_c, ea_c,
              w_nb, b_nb, w_nu0, b_nu0, w_nu1, b_nu1, w_nu2, b_nu2,
              w_ed, b_ed, w_eu, b_eu, w_2nb, b_2nb,
              x1_o, a_o, rinv_o, inv_o):
    f32 = jnp.float32
    dot = functools.partial(jnp.dot, preferred_element_type=f32)
    ea = ea_c[0] + ea_c[1]
    ea7 = ea[:, 0:7]
    cnt = ea[:, 7:8]
    inv = 1.0 / jnp.maximum(cnt, 1.0)
    xs = (xs_c[0] + xs_c[1])[:, 0:3]
    agg1 = (dot(xs, w_nb[0:3, :]) + dot(ea7, w_nb[3:10, :])
            + cnt * b_nb[...]) * inv
    z = jnp.maximum(dot(x[...], w_nu0[0:3, :]) + dot(agg1, w_nu0[3:45, :])
                    + b_nu0[...], 0.0)
    z = jnp.maximum(dot(z, w_nu1[...]) + b_nu1[...], 0.0)
    x1 = dot(z, w_nu2[...]) + b_nu2[...]
    wa = w_eu[0:128, :]
    wb = w_eu[128:256, :]
    wc = w_eu[256:384, :]
    w2a = w_2nb[0:128, :]
    w2b = w_2nb[128:256, :]
    wa_eff = w2a + dot(wa, w2b)
    wb_eff = dot(wb, w2b)
    k_eff = dot(dot(w_ed[...], wc), w2b)
    c2 = dot(dot(b_ed[...], wc) + b_eu[...], w2b) + b_2nb[...]
    a = dot(x1, wa_eff)
    b = dot(x1, wb_eff)
    rinv = (cnt * (b + c2) + dot(ea7, k_eff)) * inv
    x1_o[...] = x1
    a_o[...] = a
    rinv_o[...] = rinv
    inv_o[...] = inv


def _tc2_body(x1, as_c, rinv, inv, w_nu0, b_nu0, w_nu1, b_nu1,
              w_nu2, b_nu2, out_o):
    f32 = jnp.float32
    dot = functools.partial(jnp.dot, preferred_element_type=f32)
    agg2 = as_c[0] * inv[...] + rinv[...]
    z = jnp.maximum(dot(x1[...], w_nu0[0:128, :]) + dot(agg2, w_nu0[128:240, :])
                    + b_nu0[...], 0.0)
    z = jnp.maximum(dot(z, w_nu1[...]) + b_nu1[...], 0.0)
    out_o[...] = dot(z, w_nu2[...]) + b_nu2[...]


def _row_spec(r, w):
    return pl.BlockSpec((r, w), lambda i: (i, 0))


def _full_spec(shape):
    nd = len(shape)
    return pl.BlockSpec(shape, lambda i: (0,) * nd)


def kernel(x, edge_index, edge_attr, params):
    n = x.shape[0]
    e = edge_index.shape[1]
    n_pad = _round_up(n + 1, NS * CHUNK)
    e_pad = _round_up(e, NW * CHUNK)

    src = edge_index[0]
    dst = edge_index[1]
    pad_e = e_pad - e
    src_p = jnp.concatenate([src, jnp.zeros((pad_e,), jnp.int32)])
    dst_p = jnp.concatenate(
        [dst, n + (jnp.arange(pad_e, dtype=jnp.int32) % (n_pad - n))])
    x8 = jnp.pad(x, ((0, 0), (0, 5)))
    ea8 = jnp.pad(
        jnp.concatenate([edge_attr, jnp.ones((e, 1), jnp.float32)], axis=1),
        ((0, pad_e), (0, 0)))

    sc_a = _make_scatter_kernel(n_pad, e_pad, widths=(8, 8), n_tables=1,
                                split=(46, 34))
    z8 = jnp.zeros((CHUNK, 8), jnp.float32)
    xs_c, ea_c = sc_a(src_p, dst_p, x8, ea8, z8, z8)

    (w_nb, b_nb), = params['l1_nb']
    (wn0, bn0), (wn1, bn1), (wn2, bn2) = params['l1_nu']
    (w_ed, b_ed), = params['l1_ed']
    (w_eu, b_eu), = params['l1_eu']
    (w_2nb, b_2nb), = params['l2_nb']
    r = 1000
    grid = n // r
    row2 = lambda w: _row_spec(r, w)
    stk = lambda w: pl.BlockSpec((2, r, w), lambda i: (0, i, 0))
    tc1 = pl.pallas_call(
        _tc1_body,
        grid=(grid,),
        in_specs=[row2(3), stk(8), stk(8)]
                 + [_full_spec(s.shape) for s in (
                     w_nb, b_nb.reshape(1, -1), wn0, bn0.reshape(1, -1),
                     wn1, bn1.reshape(1, -1), wn2, bn2.reshape(1, -1),
                     w_ed, b_ed.reshape(1, -1), w_eu, b_eu.reshape(1, -1),
                     w_2nb, b_2nb.reshape(1, -1))],
        out_specs=[row2(128), row2(112), row2(112), row2(1)],
        out_shape=[jax.ShapeDtypeStruct((n, 128), jnp.float32),
                   jax.ShapeDtypeStruct((n, 112), jnp.float32),
                   jax.ShapeDtypeStruct((n, 112), jnp.float32),
                   jax.ShapeDtypeStruct((n, 1), jnp.float32)],
    )
    x1, a_mat, rinv, inv = tc1(
        x, xs_c, ea_c,
        w_nb, b_nb.reshape(1, -1), wn0, bn0.reshape(1, -1),
        wn1, bn1.reshape(1, -1), wn2, bn2.reshape(1, -1),
        w_ed, b_ed.reshape(1, -1), w_eu, b_eu.reshape(1, -1),
        w_2nb, b_2nb.reshape(1, -1))

    sc_b = _make_scatter_kernel(n_pad, e_pad, widths=(112,), n_tables=1,
                                split=(60, 20))
    z112 = jnp.zeros((CHUNK, 112), jnp.float32)
    as_c, = sc_b(src_p, dst_p, a_mat, z112)

    (w2n0, b2n0), (w2n1, b2n1), (w2n2, b2n2) = params['l2_nu']
    tc2 = pl.pallas_call(
        _tc2_body,
        grid=(grid,),
        in_specs=[row2(128), pl.BlockSpec((1, r, 112), lambda i: (0, i, 0)),
                  row2(112), row2(1)]
                 + [_full_spec(s.shape) for s in (
                     w2n0, b2n0.reshape(1, -1), w2n1, b2n1.reshape(1, -1),
                     w2n2, b2n2.reshape(1, -1))],
        out_specs=[row2(264)],
        out_shape=[jax.ShapeDtypeStruct((n, 264), jnp.float32)],
    )
    x2, = tc2(x1, as_c, rinv, inv,
              w2n0, b2n0.reshape(1, -1), w2n1, b2n1.reshape(1, -1),
              w2n2, b2n2.reshape(1, -1))
    return x2
